# baseline (speedup 1.0000x reference)
import jax
import jax.numpy as jnp
from jax import lax
from jax.experimental import pallas as pl
from jax.experimental.pallas import tpu as pltpu
from jax.experimental.pallas import tpu_sc as plsc

_NC = 2
_NS = 16
_NW = _NC * _NS

_N = 10000
_E = 320000
_DF = 128
_DE = 128
_MAXDEG = 128

_BINS = 10240
_CHUNK = _BINS // _NS
_CW = 128
_NCHUNK = _E // _CW
_CPT = _NCHUNK // _NW
_XTRA = _NCHUNK - _CPT * _NW
_UNROLL = 6


def _hist_body(edges_hbm, consts_hbm, out_hbm, idx_v, ones_v, chunk_v, shared,
               sem_c, sem_s):
    c = lax.axis_index("c")
    s = lax.axis_index("s")
    wid = c * _NS + s
    base = wid * _CPT + jnp.minimum(wid, _XTRA)

    main = _CPT * _CW
    col0 = pl.multiple_of(base * _CW, _CW)
    staged = pltpu.async_copy(edges_hbm.at[:, pl.ds(col0, main)],
                              idx_v.at[:, pl.ds(0, main)], sem_s)

    @pl.when(wid < _XTRA)
    def _():
        colx = pl.multiple_of((base + _CPT) * _CW, _CW)
        pltpu.sync_copy(edges_hbm.at[:, pl.ds(colx, _CW)],
                        idx_v.at[:, pl.ds(main, _CW)])

    pltpu.sync_copy(consts_hbm.at[pl.ds(_CHUNK, _CW)], ones_v)
    pltpu.sync_copy(consts_hbm.at[pl.ds(0, _CHUNK)],
                    shared.at[pl.ds(s * _CHUNK, _CHUNK)])
    plsc.subcore_barrier()
    staged.wait()

    def fire(j, carry):
        for k in range(_UNROLL):
            off = pl.multiple_of((j * _UNROLL + k) * _CW, _CW)
            pltpu.async_copy(ones_v,
                             shared.at[idx_v.at[1, pl.ds(off, _CW)]],
                             sem_c, add=True)
        return carry

    lax.fori_loop(0, _CPT // _UNROLL, fire, 0)

    @pl.when(wid < _XTRA)
    def _():
        pltpu.sync_copy(ones_v, shared.at[idx_v.at[1, pl.ds(main, _CW)]],
                        add=True)

    pltpu.make_async_copy(edges_hbm.at[0, pl.ds(0, main)],
                          idx_v.at[0, pl.ds(0, main)], sem_c).wait()
    plsc.subcore_barrier()

    pltpu.sync_copy(shared.at[pl.ds(s * _CHUNK, _CHUNK)], chunk_v)
    pltpu.sync_copy(chunk_v, out_hbm.at[pl.ds(c * _BINS + s * _CHUNK, _CHUNK)])


_hist = pl.kernel(
    _hist_body,
    out_type=jax.ShapeDtypeStruct((_NC * _BINS,), jnp.float32),
    mesh=plsc.VectorSubcoreMesh(core_axis_name="c", subcore_axis_name="s"),
    scratch_types=[
        pltpu.VMEM((2, (_CPT + 1) * _CW), jnp.int32),
        pltpu.VMEM((_CW,), jnp.float32),
        pltpu.VMEM((_CHUNK,), jnp.float32),
        pltpu.VMEM_SHARED((_BINS,), jnp.float32),
        pltpu.SemaphoreType.DMA,
        pltpu.SemaphoreType.DMA,
    ],
)

_RB = 1024
_RE = 5120


def _tc_copy_body(x_ref, out_ref):
    out_ref[...] = x_ref[...]


def _tc_emb_body(o1_ref, d0_ref, d1_ref, w_ref, out_ref):
    del o1_ref
    deg = jnp.reshape(d0_ref[...] + d1_ref[...], (1, _RE))
    deg_i = jnp.minimum(deg.astype(jnp.int32), _MAXDEG - 1)
    iota = lax.broadcasted_iota(jnp.int32, (_MAXDEG, _RE), 0)
    onehot = (iota == deg_i).astype(jnp.float32)
    out_ref[...] = lax.dot_general(onehot, w_ref[...],
                                   (((0,), (0,)), ((), ())),
                                   preferred_element_type=jnp.float32)


def kernel(x, edge_index, W):
    consts = jnp.concatenate([jnp.zeros((_CHUNK,), jnp.float32),
                              jnp.ones((_CW,), jnp.float32)])
    partials = _hist(edge_index, consts)
    out1 = pl.pallas_call(
        _tc_copy_body,
        grid=(pl.cdiv(_N, _RB),),
        in_specs=[pl.BlockSpec((_RB, _DF), lambda i: (i, 0))],
        out_specs=pl.BlockSpec((_RB, _DF), lambda i: (i, 0)),
        out_shape=jax.ShapeDtypeStruct((_N, _DF + _DE), jnp.float32),
    )(x)
    nblk = _BINS // _RE
    return pl.pallas_call(
        _tc_emb_body,
        grid=(pl.cdiv(_N, _RE),),
        in_specs=[
            pl.BlockSpec(memory_space=pltpu.MemorySpace.HBM),
            pl.BlockSpec((_RE,), lambda i: (i,)),
            pl.BlockSpec((_RE,), lambda i: (i + nblk,)),
            pl.BlockSpec((_MAXDEG, _DE), lambda i: (0, 0)),
        ],
        out_specs=pl.BlockSpec((_RE, _DE), lambda i: (i, 1)),
        out_shape=jax.ShapeDtypeStruct((_N, _DF + _DE), jnp.float32),
        input_output_aliases={0: 0},
    )(out1, partials, partials, W)


# speedup vs baseline: 1.1025x; 1.1025x over previous
import jax
import jax.numpy as jnp
from jax import lax
from jax.experimental import pallas as pl
from jax.experimental.pallas import tpu as pltpu
from jax.experimental.pallas import tpu_sc as plsc

_NC = 2
_NS = 16
_NW = _NC * _NS

_N = 10000
_E = 320000
_DF = 128
_DE = 128
_MAXDEG = 128

_BINS = 10240
_CHUNK = _BINS // _NS
_CW = 128
_NCHUNK = _E // _CW
_CPT = _NCHUNK // _NW
_XTRA = _NCHUNK - _CPT * _NW
_UNROLL = 6


def _hist_body(edges_hbm, out_hbm, idx_v, ones_v, chunk_v, shared,
               sem_c, sem_s):
    c = lax.axis_index("c")
    s = lax.axis_index("s")
    wid = c * _NS + s
    base = wid * _CPT + jnp.minimum(wid, _XTRA)

    main = _CPT * _CW
    col0 = pl.multiple_of(base * _CW, _CW)
    staged = pltpu.async_copy(edges_hbm.at[:, pl.ds(col0, main)],
                              idx_v.at[:, pl.ds(0, main)], sem_s)

    @pl.when(wid < _XTRA)
    def _():
        colx = pl.multiple_of((base + _CPT) * _CW, _CW)
        pltpu.sync_copy(edges_hbm.at[:, pl.ds(colx, _CW)],
                        idx_v.at[:, pl.ds(main, _CW)])

    for k in range(_CW // 16):
        ones_v[pl.ds(k * 16, 16)] = jnp.ones((16,), jnp.float32)
    for k in range(_CHUNK // 16):
        chunk_v[pl.ds(k * 16, 16)] = jnp.zeros((16,), jnp.float32)

    pltpu.sync_copy(chunk_v, shared.at[pl.ds(s * _CHUNK, _CHUNK)])
    plsc.subcore_barrier()
    staged.wait()

    def fire(j, carry):
        for k in range(_UNROLL):
            off = pl.multiple_of((j * _UNROLL + k) * _CW, _CW)
            pltpu.async_copy(ones_v,
                             shared.at[idx_v.at[1, pl.ds(off, _CW)]],
                             sem_c, add=True)
        return carry

    lax.fori_loop(0, _CPT // _UNROLL, fire, 0)

    @pl.when(wid < _XTRA)
    def _():
        pltpu.sync_copy(ones_v, shared.at[idx_v.at[1, pl.ds(main, _CW)]],
                        add=True)

    pltpu.make_async_copy(edges_hbm.at[0, pl.ds(0, main)],
                          idx_v.at[0, pl.ds(0, main)], sem_c).wait()
    plsc.subcore_barrier()

    pltpu.sync_copy(shared.at[pl.ds(s * _CHUNK, _CHUNK)], chunk_v)
    pltpu.sync_copy(chunk_v, out_hbm.at[pl.ds(c * _BINS + s * _CHUNK, _CHUNK)])


_hist = pl.kernel(
    _hist_body,
    out_type=jax.ShapeDtypeStruct((_NC * _BINS,), jnp.float32),
    mesh=plsc.VectorSubcoreMesh(core_axis_name="c", subcore_axis_name="s"),
    scratch_types=[
        pltpu.VMEM((2, (_CPT + 1) * _CW), jnp.int32),
        pltpu.VMEM((_CW,), jnp.float32),
        pltpu.VMEM((_CHUNK,), jnp.float32),
        pltpu.VMEM_SHARED((_BINS,), jnp.float32),
        pltpu.SemaphoreType.DMA,
        pltpu.SemaphoreType.DMA,
    ],
)

_RB = 1024
_RE = 5120


def _tc_copy_body(x_ref, out_ref):
    out_ref[...] = x_ref[...]


def _tc_emb_body(o1_ref, d0_ref, d1_ref, w_ref, out_ref):
    del o1_ref
    deg = jnp.reshape(d0_ref[...] + d1_ref[...], (1, _RE))
    deg_i = jnp.minimum(deg.astype(jnp.int32), _MAXDEG - 1)
    iota = lax.broadcasted_iota(jnp.int32, (_MAXDEG, _RE), 0)
    onehot = (iota == deg_i).astype(jnp.float32)
    out_ref[...] = lax.dot_general(onehot, w_ref[...],
                                   (((0,), (0,)), ((), ())),
                                   preferred_element_type=jnp.float32)


def kernel(x, edge_index, W):
    partials = _hist(edge_index)
    out1 = pl.pallas_call(
        _tc_copy_body,
        grid=(pl.cdiv(_N, _RB),),
        in_specs=[pl.BlockSpec((_RB, _DF), lambda i: (i, 0))],
        out_specs=pl.BlockSpec((_RB, _DF), lambda i: (i, 0)),
        out_shape=jax.ShapeDtypeStruct((_N, _DF + _DE), jnp.float32),
    )(x)
    nblk = _BINS // _RE
    return pl.pallas_call(
        _tc_emb_body,
        grid=(pl.cdiv(_N, _RE),),
        in_specs=[
            pl.BlockSpec(memory_space=pltpu.MemorySpace.HBM),
            pl.BlockSpec((_RE,), lambda i: (i,)),
            pl.BlockSpec((_RE,), lambda i: (i + nblk,)),
            pl.BlockSpec((_MAXDEG, _DE), lambda i: (0, 0)),
        ],
        out_specs=pl.BlockSpec((_RE, _DE), lambda i: (i, 1)),
        out_shape=jax.ShapeDtypeStruct((_N, _DF + _DE), jnp.float32),
        input_output_aliases={0: 0},
    )(out1, partials, partials, W)
